# baseline (speedup 1.0000x reference)
import functools

import jax
import jax.numpy as jnp
from jax import lax
from jax.experimental import pallas as pl
from jax.experimental.pallas import tpu as pltpu
from jax.experimental.pallas import tpu_sc as plsc

N_NODES = 10000
N_EDGES = 10000
NNZ = 160000
D = 256
DH = 128
NC = 2
NS = 16
L = 16

NPAD = 10240
ROWS_PER_TILE = NPAD // NS
NNZ_PER_TILE = NNZ // NS
CH = 80
UN = 4
NNZ_PT_PAD = 10240
NCHUNK = NNZ_PT_PAD // CH
NB = NCHUNK // UN
RCH = 80
NRCH = ROWS_PER_TILE // RCH
JUNK = N_NODES

_MESH = plsc.VectorSubcoreMesh(
    core_axis_name="c", subcore_axis_name="s", num_cores=NC, num_subcores=NS)


def _make_sc_kernels():
    out_type = [jax.ShapeDtypeStruct((NPAD, DH), jnp.float32),
                jax.ShapeDtypeStruct((NPAD, DH), jnp.float32)]
    deg_out_type = [jax.ShapeDtypeStruct((NPAD, DH), jnp.float32),
                    jax.ShapeDtypeStruct((NPAD, DH), jnp.float32)]
    scratch = [
        pltpu.VMEM_SHARED((NPAD, DH), jnp.float32),
        pltpu.VMEM((2, CH), jnp.int32),
        pltpu.VMEM((2, CH), jnp.int32),
        pltpu.VMEM((2, CH), jnp.int32),
        pltpu.VMEM((2, CH), jnp.int32),
        pltpu.VMEM((CH, DH), jnp.float32),
        pltpu.VMEM((CH, DH), jnp.float32),
        pltpu.VMEM((CH, DH), jnp.float32),
        pltpu.VMEM((CH, DH), jnp.float32),
        pltpu.SemaphoreType.DMA,
        pltpu.SemaphoreType.DMA,
        pltpu.SemaphoreType.DMA,
        pltpu.SemaphoreType.DMA,
        pltpu.SemaphoreType.DMA,
        pltpu.SemaphoreType.DMA,
        pltpu.SemaphoreType.DMA,
        pltpu.SemaphoreType.DMA,
    ]

    def _common(refs):
        acc = refs[0]
        PAIR = refs[1:5]
        ROW = refs[5:9]
        GS = refs[9:13]
        IS = refs[13:17]
        c = lax.axis_index("c")
        s = lax.axis_index("s")
        sl0 = pl.ds(0, RCH)

        def fill_const(ref, val):
            v = jnp.full((L,), val, jnp.float32)

            def fb(r, carry):
                for j in range(DH // L):
                    ref[r, pl.ds(j * L, L)] = v
                return carry
            lax.fori_loop(0, CH, fb, 0)

        def zero_acc():
            fill_const(ROW[1], 0.0)
            for k in range(NRCH):
                pltpu.sync_copy(
                    ROW[1].at[sl0],
                    acc.at[pl.ds(s * ROWS_PER_TILE + k * RCH, RCH)])

        def hist_pass(ps):
            NH = NCHUNK // 2
            pltpu.sync_copy(ps.at[0], PAIR[0])
            pltpu.async_copy(ps.at[1], PAIR[1], IS[1])

            def hb(j, carry):
                @pl.when(j > 0)
                def _():
                    pltpu.make_async_copy(ps.at[0], PAIR[0], IS[0]).wait()
                pltpu.sync_copy(ROW[0], acc.at[PAIR[0].at[1]], add=True)

                @pl.when(j < NH - 1)
                def _():
                    pltpu.async_copy(ps.at[2 * j + 2], PAIR[0], IS[0])
                pltpu.make_async_copy(ps.at[0], PAIR[1], IS[1]).wait()
                pltpu.sync_copy(ROW[0], acc.at[PAIR[1].at[1]], add=True)

                @pl.when(j < NH - 1)
                def _():
                    pltpu.async_copy(ps.at[2 * j + 3], PAIR[1], IS[1])
                return carry
            lax.fori_loop(0, NH, hb, 0)

        def dump_deg(table):
            for k in range(NRCH):
                sl = pl.ds(s * ROWS_PER_TILE + k * RCH, RCH)
                pltpu.sync_copy(acc.at[sl], ROW[1].at[sl0])
                pltpu.sync_copy(ROW[1].at[sl0], table.at[sl])

        def seg_pass(src_hbm, ps):
            def g_start(p, x):
                pltpu.async_copy(src_hbm.at[PAIR[p].at[0]], ROW[x], GS[x])

            def g_wait(p, x):
                pltpu.make_async_copy(
                    src_hbm.at[PAIR[p].at[0]], ROW[x], GS[x]).wait()

            def i_start(i, p):
                pltpu.async_copy(ps.at[i], PAIR[p], IS[p])

            def i_wait(p):
                pltpu.make_async_copy(ps.at[0], PAIR[p], IS[p]).wait()

            pltpu.sync_copy(ps.at[0], PAIR[0])
            pltpu.sync_copy(ps.at[1], PAIR[1])
            pltpu.sync_copy(ps.at[2], PAIR[2])
            g_start(0, 0)
            g_start(1, 1)
            g_start(2, 2)
            i_start(3, 3)

            def sb(k, carry):
                base = UN * k
                for t in range(UN):
                    i = base + t
                    x = t % 4
                    g_wait(x, x)
                    pltpu.sync_copy(ROW[x], acc.at[PAIR[x].at[1]], add=True)

                    @pl.when(i + 4 < NCHUNK)
                    def _():
                        i_start(i + 4, x)

                    @pl.when(i + 3 < NCHUNK)
                    def _():
                        i_wait((t + 3) % 4)
                        g_start((t + 3) % 4, (t + 3) % 4)
                return carry
            lax.fori_loop(0, NB, sb, 0)

        def scale_write(table, out_hbm):
            for k in range(NRCH):
                sl = pl.ds(s * ROWS_PER_TILE + k * RCH, RCH)
                pltpu.sync_copy(acc.at[sl], ROW[0].at[sl0])
                pltpu.sync_copy(table.at[sl], ROW[1].at[sl0])

                def rb(r, carry):
                    dv = ROW[1][r, pl.ds(0, L)]
                    inv = jnp.where(dv > 0.0, 1.0 / dv, 0.0)
                    for j in range(DH // L):
                        cs = pl.ds(j * L, L)
                        ROW[0][r, cs] = ROW[0][r, cs] * inv
                    return carry
                lax.fori_loop(0, RCH, rb, 0)
                pltpu.sync_copy(ROW[0].at[sl0], out_hbm.at[sl])

        return (c, s, fill_const, zero_acc, hist_pass, dump_deg,
                seg_pass, scale_write)

    def deg_body(pairs1, pairs2, ndeg_out, edeg_out, *refs):
        (c, s, fill_const, zero_acc, hist_pass, dump_deg,
         seg_pass, scale_write) = _common(refs)
        p1 = pairs1.at[s]
        p2 = pairs2.at[s]
        fill_const(refs[5], 1.0)
        zero_acc()
        plsc.subcore_barrier()

        @pl.when(c == 0)
        def _():
            hist_pass(p1)

        @pl.when(c == 1)
        def _():
            hist_pass(p2)
        plsc.subcore_barrier()

        @pl.when(c == 0)
        def _():
            dump_deg(edeg_out)

        @pl.when(c == 1)
        def _():
            dump_deg(ndeg_out)

    def layer_body(xl_a, xl_b, pairs1, pairs2, ndeg, edeg,
                   out_a, out_b, *refs):
        (c, s, fill_const, zero_acc, hist_pass, dump_deg,
         seg_pass, scale_write) = _common(refs)
        p1 = pairs1.at[s]
        p2 = pairs2.at[s]
        zero_acc()
        plsc.subcore_barrier()

        def run_half(xl, out_h):
            seg_pass(xl, p1)
            plsc.subcore_barrier()
            scale_write(edeg, out_h)
            zero_acc()
            plsc.subcore_barrier()
            seg_pass(out_h, p2)
            plsc.subcore_barrier()
            scale_write(ndeg, out_h)

        @pl.when(c == 0)
        def _():
            run_half(xl_a, out_a)

        @pl.when(c == 1)
        def _():
            run_half(xl_b, out_b)

    deg_k = pl.kernel(deg_body, out_type=deg_out_type, mesh=_MESH,
                      scratch_types=scratch)
    layer_k = pl.kernel(layer_body, out_type=out_type, mesh=_MESH,
                        scratch_types=scratch)
    return deg_k, layer_k


_sc_degrees, _sc_layer = _make_sc_kernels()


_BLK = 1000
_GRID = (N_NODES // _BLK,)


def _dot_t(x, w):
    return lax.dot_general(x, w, (((1,), (1,)), ((), ())),
                           preferred_element_type=jnp.float32)


def _tc_lin(X, W):
    def body(x_ref, w_ref, oa_ref, ob_ref):
        y = _dot_t(x_ref[...], w_ref[...])
        oa_ref[...] = y[:, :DH]
        ob_ref[...] = y[:, DH:]

    return pl.pallas_call(
        body,
        grid=_GRID,
        in_specs=[pl.BlockSpec((_BLK, D), lambda i: (i, 0)),
                  pl.BlockSpec((D, D), lambda i: (0, 0))],
        out_specs=[pl.BlockSpec((_BLK, DH), lambda i: (i, 0)),
                   pl.BlockSpec((_BLK, DH), lambda i: (i, 0))],
        out_shape=[jax.ShapeDtypeStruct((N_NODES, DH), jnp.float32)] * 2,
    )(X, W)


def _epilogue(ya, yb, b, g, beta):
    h = jnp.concatenate([ya, yb], axis=1) + b
    h = jnp.where(h >= 0.0, h, 0.01 * h)
    mu = jnp.mean(h, axis=1, keepdims=True)
    d = h - mu
    var = jnp.mean(d * d, axis=1, keepdims=True)
    return d * lax.rsqrt(var + 1e-5) * g + beta


def _tc_mid(ya, yb, b, g, beta, W):
    def body(ya_ref, yb_ref, b_ref, g_ref, be_ref, w_ref,
             h_ref, oa_ref, ob_ref):
        hn = _epilogue(ya_ref[...], yb_ref[...],
                       b_ref[...], g_ref[...], be_ref[...])
        h_ref[...] = hn
        y = _dot_t(hn, w_ref[...])
        oa_ref[...] = y[:, :DH]
        ob_ref[...] = y[:, DH:]

    vec = pl.BlockSpec((1, D), lambda i: (0, 0))
    return pl.pallas_call(
        body,
        grid=_GRID,
        in_specs=[pl.BlockSpec((_BLK, DH), lambda i: (i, 0)),
                  pl.BlockSpec((_BLK, DH), lambda i: (i, 0)),
                  vec, vec, vec,
                  pl.BlockSpec((D, D), lambda i: (0, 0))],
        out_specs=[pl.BlockSpec((_BLK, D), lambda i: (i, 0)),
                   pl.BlockSpec((_BLK, DH), lambda i: (i, 0)),
                   pl.BlockSpec((_BLK, DH), lambda i: (i, 0))],
        out_shape=[jax.ShapeDtypeStruct((N_NODES, D), jnp.float32),
                   jax.ShapeDtypeStruct((N_NODES, DH), jnp.float32),
                   jax.ShapeDtypeStruct((N_NODES, DH), jnp.float32)],
    )(ya, yb, b, g, beta, W)


def _tc_final(ya, yb, b, g, beta, X, h1):
    def body(ya_ref, yb_ref, b_ref, g_ref, be_ref, x_ref, h1_ref, o_ref):
        h2 = _epilogue(ya_ref[...], yb_ref[...],
                       b_ref[...], g_ref[...], be_ref[...])
        o_ref[...] = (x_ref[...] + h1_ref[...] + h2) * (1.0 / 3.0)

    vec = pl.BlockSpec((1, D), lambda i: (0, 0))
    return pl.pallas_call(
        body,
        grid=_GRID,
        in_specs=[pl.BlockSpec((_BLK, DH), lambda i: (i, 0)),
                  pl.BlockSpec((_BLK, DH), lambda i: (i, 0)),
                  vec, vec, vec,
                  pl.BlockSpec((_BLK, D), lambda i: (i, 0)),
                  pl.BlockSpec((_BLK, D), lambda i: (i, 0))],
        out_specs=pl.BlockSpec((_BLK, D), lambda i: (i, 0)),
        out_shape=jax.ShapeDtypeStruct((N_NODES, D), jnp.float32),
    )(ya, yb, b, g, beta, X, h1)


def kernel(X, A, W0, b0, g0, beta0, W1, b1, g1, beta1):
    pad = ((0, 0), (0, NNZ_PT_PAD - NNZ_PER_TILE))
    n2 = A[0].reshape(NS, NNZ_PER_TILE)
    e2 = A[1].reshape(NS, NNZ_PER_TILE)
    n0 = jnp.pad(n2, pad).reshape(NS, NCHUNK, CH)
    nj = jnp.pad(n2, pad, constant_values=JUNK).reshape(NS, NCHUNK, CH)
    ej = jnp.pad(e2, pad, constant_values=JUNK).reshape(NS, NCHUNK, CH)
    pairs1 = jnp.stack([n0, ej], axis=2)
    pairs2 = jnp.stack([ej, nj], axis=2)
    b0r, g0r, be0r = (v.reshape(1, D) for v in (b0, g0, beta0))
    b1r, g1r, be1r = (v.reshape(1, D) for v in (b1, g1, beta1))

    ndeg, edeg = _sc_degrees(pairs1, pairs2)
    xa0, xb0 = _tc_lin(X, W0)
    oa0, ob0 = _sc_layer(xa0, xb0, pairs1, pairs2, ndeg, edeg)
    h1, xa1, xb1 = _tc_mid(oa0, ob0, b0r, g0r, be0r, W1)
    oa1, ob1 = _sc_layer(xa1, xb1, pairs1, pairs2, ndeg, edeg)
    return _tc_final(oa1, ob1, b1r, g1r, be1r, X, h1)


# speedup vs baseline: 1.4426x; 1.4426x over previous
import jax
import jax.numpy as jnp
from jax import lax
from jax.experimental import pallas as pl
from jax.experimental.pallas import tpu as pltpu
from jax.experimental.pallas import tpu_sc as plsc

N_NODES = 10000
N_EDGES = 10000
NNZ = 160000
D = 256
DH = 128
NC = 2
NS = 16
L = 16

NPAD = 10240
ROWS_PER_TILE = NPAD // NS
NNZ_PER_TILE = NNZ // NS
CH = 112
UN = 6
NNZ_PT_PAD = 10080
NCHUNK = NNZ_PT_PAD // CH
NB = NCHUNK // UN
RCH = 80
NRCH = ROWS_PER_TILE // RCH
JUNK = N_NODES

_MESH = plsc.VectorSubcoreMesh(
    core_axis_name="c", subcore_axis_name="s", num_cores=NC, num_subcores=NS)


def _make_sc_kernels():
    out_type = [jax.ShapeDtypeStruct((NPAD, DH), jnp.float32),
                jax.ShapeDtypeStruct((NPAD, DH), jnp.float32)]
    deg_out_type = [jax.ShapeDtypeStruct((NPAD, DH), jnp.float32),
                    jax.ShapeDtypeStruct((NPAD, DH), jnp.float32)]
    scratch = [
        pltpu.VMEM_SHARED((NPAD, DH), jnp.float32),
        pltpu.VMEM((2, CH), jnp.int32),
        pltpu.VMEM((2, CH), jnp.int32),
        pltpu.VMEM((2, CH), jnp.int32),
        pltpu.VMEM((2, CH), jnp.int32),
        pltpu.VMEM((CH, DH), jnp.float32),
        pltpu.VMEM((CH, DH), jnp.float32),
        pltpu.VMEM((CH, DH), jnp.float32),
        pltpu.SemaphoreType.DMA,
        pltpu.SemaphoreType.DMA,
        pltpu.SemaphoreType.DMA,
        pltpu.SemaphoreType.DMA,
        pltpu.SemaphoreType.DMA,
        pltpu.SemaphoreType.DMA,
        pltpu.SemaphoreType.DMA,
    ]

    def _common(refs):
        acc = refs[0]
        PAIR = refs[1:5]
        ROW = refs[5:8]
        GS = refs[8:11]
        IS = refs[11:15]
        c = lax.axis_index("c")
        s = lax.axis_index("s")
        sl0 = pl.ds(0, RCH)

        def fill_const(ref, val):
            v = jnp.full((L,), val, jnp.float32)

            def fb(r, carry):
                for j in range(DH // L):
                    ref[r, pl.ds(j * L, L)] = v
                return carry
            lax.fori_loop(0, CH, fb, 0)

        def zero_acc():
            fill_const(ROW[1], 0.0)
            for k in range(NRCH):
                pltpu.sync_copy(
                    ROW[1].at[sl0],
                    acc.at[pl.ds(s * ROWS_PER_TILE + k * RCH, RCH)])

        def hist_pass(ps):
            NH = NCHUNK // 2
            pltpu.sync_copy(ps.at[0], PAIR[0])
            pltpu.async_copy(ps.at[1], PAIR[1], IS[1])

            def hb(j, carry):
                @pl.when(j > 0)
                def _():
                    pltpu.make_async_copy(ps.at[0], PAIR[0], IS[0]).wait()
                pltpu.sync_copy(ROW[0], acc.at[PAIR[0].at[1]], add=True)

                @pl.when(j < NH - 1)
                def _():
                    pltpu.async_copy(ps.at[2 * j + 2], PAIR[0], IS[0])
                pltpu.make_async_copy(ps.at[0], PAIR[1], IS[1]).wait()
                pltpu.sync_copy(ROW[0], acc.at[PAIR[1].at[1]], add=True)

                @pl.when(j < NH - 1)
                def _():
                    pltpu.async_copy(ps.at[2 * j + 3], PAIR[1], IS[1])
                return carry
            lax.fori_loop(0, NH, hb, 0)

        def dump_deg(table):
            for k in range(NRCH):
                sl = pl.ds(s * ROWS_PER_TILE + k * RCH, RCH)
                pltpu.sync_copy(acc.at[sl], ROW[1].at[sl0])
                pltpu.sync_copy(ROW[1].at[sl0], table.at[sl])

        def seg_pass(src_hbm, ps):
            def g_start(p, x):
                pltpu.async_copy(src_hbm.at[PAIR[p].at[0]], ROW[x], GS[x])

            def g_wait(p, x):
                pltpu.make_async_copy(
                    src_hbm.at[PAIR[p].at[0]], ROW[x], GS[x]).wait()

            def i_start(i, p):
                pltpu.async_copy(ps.at[i], PAIR[p], IS[p])

            def i_wait(p):
                pltpu.make_async_copy(ps.at[0], PAIR[p], IS[p]).wait()

            pltpu.sync_copy(ps.at[0], PAIR[0])
            pltpu.sync_copy(ps.at[1], PAIR[1])
            g_start(0, 0)
            g_start(1, 1)
            i_start(2, 2)

            def sb(k, carry):
                base = UN * k
                for t in range(UN):
                    i = base + t
                    x = t % 3
                    g_wait(x, x)
                    pltpu.sync_copy(ROW[x], acc.at[PAIR[x].at[1]], add=True)

                    @pl.when(i + 3 < NCHUNK)
                    def _():
                        i_start(i + 3, x)

                    @pl.when(i + 2 < NCHUNK)
                    def _():
                        i_wait((t + 2) % 3)
                        g_start((t + 2) % 3, (t + 2) % 3)
                return carry
            lax.fori_loop(0, NB, sb, 0)

        def scale_write(table, out_hbm):
            for k in range(NRCH):
                sl = pl.ds(s * ROWS_PER_TILE + k * RCH, RCH)
                pltpu.sync_copy(acc.at[sl], ROW[0].at[sl0])
                pltpu.sync_copy(table.at[sl], ROW[1].at[sl0])

                def rb(r, carry):
                    dv = ROW[1][r, pl.ds(0, L)]
                    inv = jnp.where(dv > 0.0, 1.0 / dv, 0.0)
                    for j in range(DH // L):
                        cs = pl.ds(j * L, L)
                        ROW[0][r, cs] = ROW[0][r, cs] * inv
                    return carry
                lax.fori_loop(0, RCH, rb, 0)
                pltpu.sync_copy(ROW[0].at[sl0], out_hbm.at[sl])

        return (c, s, fill_const, zero_acc, hist_pass, dump_deg,
                seg_pass, scale_write)

    def deg_body(pairs1, pairs2, ndeg_out, edeg_out, *refs):
        (c, s, fill_const, zero_acc, hist_pass, dump_deg,
         seg_pass, scale_write) = _common(refs)
        p1 = pairs1.at[s]
        p2 = pairs2.at[s]
        fill_const(refs[5], 1.0)
        zero_acc()
        plsc.subcore_barrier()

        @pl.when(c == 0)
        def _():
            hist_pass(p1)

        @pl.when(c == 1)
        def _():
            hist_pass(p2)
        plsc.subcore_barrier()

        @pl.when(c == 0)
        def _():
            dump_deg(edeg_out)

        @pl.when(c == 1)
        def _():
            dump_deg(ndeg_out)

    def layer_body(xl_a, xl_b, pairs1, pairs2, ndeg, edeg,
                   out_a, out_b, *refs):
        (c, s, fill_const, zero_acc, hist_pass, dump_deg,
         seg_pass, scale_write) = _common(refs)
        p1 = pairs1.at[s]
        p2 = pairs2.at[s]
        zero_acc()
        plsc.subcore_barrier()

        def run_half(xl, out_h):
            seg_pass(xl, p1)
            plsc.subcore_barrier()
            scale_write(edeg, out_h)
            zero_acc()
            plsc.subcore_barrier()
            seg_pass(out_h, p2)
            plsc.subcore_barrier()
            scale_write(ndeg, out_h)

        @pl.when(c == 0)
        def _():
            run_half(xl_a, out_a)

        @pl.when(c == 1)
        def _():
            run_half(xl_b, out_b)

    deg_k = pl.kernel(deg_body, out_type=deg_out_type, mesh=_MESH,
                      scratch_types=scratch)
    layer_k = pl.kernel(layer_body, out_type=out_type, mesh=_MESH,
                        scratch_types=scratch)
    return deg_k, layer_k


_sc_degrees, _sc_layer = _make_sc_kernels()


_BLK = 1000
_GRID = (N_NODES // _BLK,)


def _dot_t(x, w):
    return lax.dot_general(x, w, (((1,), (1,)), ((), ())),
                           preferred_element_type=jnp.float32)


def _tc_lin(X, W):
    def body(x_ref, w_ref, oa_ref, ob_ref):
        y = _dot_t(x_ref[...], w_ref[...])
        oa_ref[...] = y[:, :DH]
        ob_ref[...] = y[:, DH:]

    return pl.pallas_call(
        body,
        grid=_GRID,
        in_specs=[pl.BlockSpec((_BLK, D), lambda i: (i, 0)),
                  pl.BlockSpec((D, D), lambda i: (0, 0))],
        out_specs=[pl.BlockSpec((_BLK, DH), lambda i: (i, 0)),
                   pl.BlockSpec((_BLK, DH), lambda i: (i, 0))],
        out_shape=[jax.ShapeDtypeStruct((N_NODES, DH), jnp.float32)] * 2,
    )(X, W)


def _epilogue(ya, yb, b, g, beta):
    h = jnp.concatenate([ya, yb], axis=1) + b
    h = jnp.where(h >= 0.0, h, 0.01 * h)
    mu = jnp.mean(h, axis=1, keepdims=True)
    d = h - mu
    var = jnp.mean(d * d, axis=1, keepdims=True)
    return d * lax.rsqrt(var + 1e-5) * g + beta


def _tc_mid(ya, yb, b, g, beta, W):
    def body(ya_ref, yb_ref, b_ref, g_ref, be_ref, w_ref,
             h_ref, oa_ref, ob_ref):
        hn = _epilogue(ya_ref[...], yb_ref[...],
                       b_ref[...], g_ref[...], be_ref[...])
        h_ref[...] = hn
        y = _dot_t(hn, w_ref[...])
        oa_ref[...] = y[:, :DH]
        ob_ref[...] = y[:, DH:]

    vec = pl.BlockSpec((1, D), lambda i: (0, 0))
    return pl.pallas_call(
        body,
        grid=_GRID,
        in_specs=[pl.BlockSpec((_BLK, DH), lambda i: (i, 0)),
                  pl.BlockSpec((_BLK, DH), lambda i: (i, 0)),
                  vec, vec, vec,
                  pl.BlockSpec((D, D), lambda i: (0, 0))],
        out_specs=[pl.BlockSpec((_BLK, D), lambda i: (i, 0)),
                   pl.BlockSpec((_BLK, DH), lambda i: (i, 0)),
                   pl.BlockSpec((_BLK, DH), lambda i: (i, 0))],
        out_shape=[jax.ShapeDtypeStruct((N_NODES, D), jnp.float32),
                   jax.ShapeDtypeStruct((N_NODES, DH), jnp.float32),
                   jax.ShapeDtypeStruct((N_NODES, DH), jnp.float32)],
    )(ya, yb, b, g, beta, W)


def _tc_final(ya, yb, b, g, beta, X, h1):
    def body(ya_ref, yb_ref, b_ref, g_ref, be_ref, x_ref, h1_ref, o_ref):
        h2 = _epilogue(ya_ref[...], yb_ref[...],
                       b_ref[...], g_ref[...], be_ref[...])
        o_ref[...] = (x_ref[...] + h1_ref[...] + h2) * (1.0 / 3.0)

    vec = pl.BlockSpec((1, D), lambda i: (0, 0))
    return pl.pallas_call(
        body,
        grid=_GRID,
        in_specs=[pl.BlockSpec((_BLK, DH), lambda i: (i, 0)),
                  pl.BlockSpec((_BLK, DH), lambda i: (i, 0)),
                  vec, vec, vec,
                  pl.BlockSpec((_BLK, D), lambda i: (i, 0)),
                  pl.BlockSpec((_BLK, D), lambda i: (i, 0))],
        out_specs=pl.BlockSpec((_BLK, D), lambda i: (i, 0)),
        out_shape=jax.ShapeDtypeStruct((N_NODES, D), jnp.float32),
    )(ya, yb, b, g, beta, X, h1)


def kernel(X, A, W0, b0, g0, beta0, W1, b1, g1, beta1):
    pad = ((0, 0), (0, NNZ_PT_PAD - NNZ_PER_TILE))
    n2 = A[0].reshape(NS, NNZ_PER_TILE)
    e2 = A[1].reshape(NS, NNZ_PER_TILE)
    n0 = jnp.pad(n2, pad).reshape(NS, NCHUNK, CH)
    nj = jnp.pad(n2, pad, constant_values=JUNK).reshape(NS, NCHUNK, CH)
    ej = jnp.pad(e2, pad, constant_values=JUNK).reshape(NS, NCHUNK, CH)
    pairs1 = jnp.stack([n0, ej], axis=2)
    pairs2 = jnp.stack([ej, nj], axis=2)
    b0r, g0r, be0r = (v.reshape(1, D) for v in (b0, g0, beta0))
    b1r, g1r, be1r = (v.reshape(1, D) for v in (b1, g1, beta1))

    ndeg, edeg = _sc_degrees(pairs1, pairs2)
    xa0, xb0 = _tc_lin(X, W0)
    oa0, ob0 = _sc_layer(xa0, xb0, pairs1, pairs2, ndeg, edeg)
    h1, xa1, xb1 = _tc_mid(oa0, ob0, b0r, g0r, be0r, W1)
    oa1, ob1 = _sc_layer(xa1, xb1, pairs1, pairs2, ndeg, edeg)
    return _tc_final(oa1, ob1, b1r, g1r, be1r, X, h1)
